# baseline (speedup 1.0000x reference)
import functools

import jax
import jax.numpy as jnp
from jax import lax
from jax.experimental import pallas as pl
from jax.experimental.pallas import tpu as pltpu
from jax.experimental.pallas import tpu_sc as plsc

_B = 16384
_V = 100000
_D = 16
_NDC = 20
_NWC = 6
_NT = _NDC + _NWC
_VL = _V * _D // 256

_NC = 2
_NS = 16
_BPT = _B // (_NC * _NS)
_CH = 128
_NCH = _BPT // _CH

_sc_mesh = plsc.VectorSubcoreMesh(
    core_axis_name="c", subcore_axis_name="s",
    num_cores=_NC, num_subcores=_NS,
)


@functools.partial(
    pl.kernel,
    out_type=tuple(
        jax.ShapeDtypeStruct((_D, _B), jnp.float32) for _ in range(_NT)
    ),
    mesh=_sc_mesh,
    compiler_params=pltpu.CompilerParams(needs_layout_passes=False),
    scratch_types=[
        pltpu.VMEM((2, _BPT), jnp.int32),
        pltpu.VMEM((_BPT,), jnp.int32),
        pltpu.VMEM((_BPT, 128), jnp.int32),
        pltpu.VMEM((_D, _BPT), jnp.float32),
        pltpu.SemaphoreType.DMA((_NCH,)),
        pltpu.SemaphoreType.DMA,
    ],
)
def _sc_gather(*refs):
    idx_refs = refs[0:_NT]
    tab_refs = refs[_NT:2 * _NT]
    out_refs = refs[2 * _NT:3 * _NT]
    idx_v, row_v, lines, stage, sem_g, sem_i = refs[-6:]

    cid = lax.axis_index("c")
    sid = lax.axis_index("s")
    base = (cid * _NS + sid) * _BPT
    lanes = lax.iota(jnp.int32, 16)

    pltpu.async_copy(idx_refs[0].at[pl.ds(base, _BPT)], idx_v.at[0], sem_i)

    def do_table(t):
        p = t % 2
        pltpu.make_async_copy(
            idx_refs[t].at[pl.ds(base, _BPT)], idx_v.at[p], sem_i).wait()

        def rid(k, _):
            iv = idx_v[p, pl.ds(k * 16, 16)]
            row_v[pl.ds(k * 16, 16)] = iv >> 4
            return ()
        lax.fori_loop(0, _BPT // 16, rid, (), unroll=False)

        def gfire(j, _):
            pltpu.async_copy(
                tab_refs[t].at[row_v.at[pl.ds(j * _CH, _CH)]],
                lines.at[pl.ds(j * _CH, _CH), :],
                sem_g.at[j],
            )
            return ()
        lax.fori_loop(0, _NCH, gfire, (), unroll=False)

        if t + 1 < _NT:
            pltpu.async_copy(
                idx_refs[t + 1].at[pl.ds(base, _BPT)], idx_v.at[1 - p], sem_i)

        def chunk(j, _):
            pltpu.make_async_copy(
                tab_refs[t].at[row_v.at[pl.ds(0, _CH)]],
                lines.at[pl.ds(0, _CH), :],
                sem_g.at[j],
            ).wait()

            def extract(gg, _):
                g = j * (_CH // 16) + gg
                iv = idx_v[p, pl.ds(g * 16, 16)]
                rows16 = g * 16 + lanes
                offs = (iv & 15) * 8

                def per_w(w, _):
                    wd = plsc.load_gather(lines, [rows16, offs + w])
                    stage[2 * w, pl.ds(g * 16, 16)] = plsc.bitcast(
                        wd << 16, jnp.float32)
                    stage[2 * w + 1, pl.ds(g * 16, 16)] = plsc.bitcast(
                        wd & jnp.int32(-65536), jnp.float32)
                    return ()
                lax.fori_loop(0, _D // 2, per_w, (), unroll=False)
                return ()
            lax.fori_loop(0, _CH // 16, extract, (), unroll=False)
            return ()
        lax.fori_loop(0, _NCH, chunk, (), unroll=False)

        pltpu.sync_copy(stage, out_refs[t].at[:, pl.ds(base, _BPT)])

    for t in range(_NT):
        do_table(t)


_BLK = 2048


def _tc_body(*refs):
    feats = refs[0:_NT]
    (dn0_ref, dn1_ref, wn0_ref, wn1_ref,
     w0_ref, w0n_ref, b0_ref, w1_ref, b1_ref, w2_ref, b2_ref,
     ww_ref, wnw_ref, wh_ref, ob_ref, out_ref) = refs[_NT:]
    dnums = (((0,), (0,)), ((), ()))
    h = b0_ref[:] + dn0_ref[:] * w0n_ref[0:1, :] + dn1_ref[:] * w0n_ref[1:2, :]
    for t in range(_NDC):
        h = h + lax.dot_general(feats[t][:], w0_ref[pl.ds(t * _D, _D), :],
                                dnums, preferred_element_type=jnp.float32)
    h = jax.nn.relu(h)
    h = jax.nn.relu(
        jnp.dot(h, w1_ref[:], preferred_element_type=jnp.float32) + b1_ref[:])
    h = jax.nn.relu(
        jnp.dot(h, w2_ref[:], preferred_element_type=jnp.float32) + b2_ref[:])
    acc = jnp.sum(h * wh_ref[:], axis=1, keepdims=True)
    for t in range(_NWC):
        acc = acc + lax.dot_general(feats[_NDC + t][:],
                                    ww_ref[pl.ds(t * _D, _D), :],
                                    dnums, preferred_element_type=jnp.float32)
    acc = acc + wn0_ref[:] * wnw_ref[:, 0:1] + wn1_ref[:] * wnw_ref[:, 1:2]
    out_ref[:] = jax.nn.sigmoid(acc + ob_ref[:])


def _tc_forward(feats, dn0, dn1, wn0, wn1,
                w0e, w0n, b0, w1, b1, w2, b2, ww, wnw, wh, ob):
    grid = _B // _BLK
    colT = lambda i: (0, i)
    row = lambda i: (i, 0)
    rep = lambda i: (0, 0)
    return pl.pallas_call(
        _tc_body,
        grid=(grid,),
        in_specs=(
            [pl.BlockSpec((_D, _BLK), colT) for _ in range(_NT)] +
            [
                pl.BlockSpec((_BLK, 1), row),
                pl.BlockSpec((_BLK, 1), row),
                pl.BlockSpec((_BLK, 1), row),
                pl.BlockSpec((_BLK, 1), row),
                pl.BlockSpec((_NDC * _D, 128), rep),
                pl.BlockSpec((2, 128), rep),
                pl.BlockSpec((1, 128), rep),
                pl.BlockSpec((128, 64), rep),
                pl.BlockSpec((1, 64), rep),
                pl.BlockSpec((64, 32), rep),
                pl.BlockSpec((1, 32), rep),
                pl.BlockSpec((_NWC * _D, 1), rep),
                pl.BlockSpec((1, 2), rep),
                pl.BlockSpec((1, 32), rep),
                pl.BlockSpec((1, 1), rep),
            ]
        ),
        out_specs=pl.BlockSpec((_BLK, 1), row),
        out_shape=jax.ShapeDtypeStruct((_B, 1), jnp.float32),
    )(*feats, dn0, dn1, wn0, wn1,
      w0e, w0n, b0, w1, b1, w2, b2, ww, wnw, wh, ob)


def kernel(dcat0, dcat1, dcat2, dcat3, dcat4, dcat5, dcat6, dcat7, dcat8, dcat9, dcat10, dcat11, dcat12, dcat13, dcat14, dcat15, dcat16, dcat17, dcat18, dcat19, dnum0, dnum1, wcat0, wcat1, wcat2, wcat3, wcat4, wcat5, wnum0, wnum1, deep_emb_0, deep_emb_1, deep_emb_2, deep_emb_3, deep_emb_4, deep_emb_5, deep_emb_6, deep_emb_7, deep_emb_8, deep_emb_9, deep_emb_10, deep_emb_11, deep_emb_12, deep_emb_13, deep_emb_14, deep_emb_15, deep_emb_16, deep_emb_17, deep_emb_18, deep_emb_19, wide_emb_0, wide_emb_1, wide_emb_2, wide_emb_3, wide_emb_4, wide_emb_5, mlp_W0, mlp_b0, mlp_W1, mlp_b1, mlp_W2, mlp_b2, out_W, out_b):
    idx = (dcat0, dcat1, dcat2, dcat3, dcat4, dcat5, dcat6, dcat7, dcat8,
           dcat9, dcat10, dcat11, dcat12, dcat13, dcat14, dcat15, dcat16,
           dcat17, dcat18, dcat19, wcat0, wcat1, wcat2, wcat3, wcat4, wcat5)
    tabs = (deep_emb_0, deep_emb_1, deep_emb_2, deep_emb_3, deep_emb_4,
            deep_emb_5, deep_emb_6, deep_emb_7, deep_emb_8, deep_emb_9,
            deep_emb_10, deep_emb_11, deep_emb_12, deep_emb_13, deep_emb_14,
            deep_emb_15, deep_emb_16, deep_emb_17, deep_emb_18, deep_emb_19,
            wide_emb_0, wide_emb_1, wide_emb_2, wide_emb_3, wide_emb_4,
            wide_emb_5)
    tabs_f = tuple(
        jax.lax.reshape(
            jax.lax.bitcast_convert_type(
                jax.lax.convert_element_type(t, jnp.bfloat16).reshape(
                    _V, _D // 2, 2), jnp.int32),
            (_VL, 128))
        for t in tabs)
    feats = _sc_gather(*idx, *tabs_f)

    w0e = mlp_W0[:_NDC * _D]
    w0n = mlp_W0[_NDC * _D:]
    b0 = mlp_b0.reshape(1, -1)
    b1 = mlp_b1.reshape(1, -1)
    b2 = mlp_b2.reshape(1, -1)
    ww = out_W[:_NWC * _D]
    wnw = out_W[_NWC * _D:_NWC * _D + 2].reshape(1, 2)
    wh = out_W[_NWC * _D + 2:].reshape(1, 32)
    ob = out_b.reshape(1, 1)
    return _tc_forward(feats, dnum0, dnum1, wnum0, wnum1,
                       w0e, w0n, b0, mlp_W1, b1, mlp_W2, b2, ww, wnw, wh, ob)


# speedup vs baseline: 2.6918x; 2.6918x over previous
import functools

import jax
import jax.numpy as jnp
from jax import lax
from jax.experimental import pallas as pl
from jax.experimental.pallas import tpu as pltpu
from jax.experimental.pallas import tpu_sc as plsc

_B = 16384
_V = 100000
_D = 16
_NDC = 20
_NWC = 6
_NT = _NDC + _NWC
_VL = _V * _D // 128

_NC = 2
_NS = 16
_BPT = _B // (_NC * _NS)
_CH = 128
_NCH = _BPT // _CH

_sc_mesh = plsc.VectorSubcoreMesh(
    core_axis_name="c", subcore_axis_name="s",
    num_cores=_NC, num_subcores=_NS,
)


def _make_sc_gather(nt):
  @functools.partial(
      pl.kernel,
      out_type=tuple(
          jax.ShapeDtypeStruct((_D, _B), jnp.float32) for _ in range(nt)
      ),
      mesh=_sc_mesh,
      compiler_params=pltpu.CompilerParams(needs_layout_passes=False),
      scratch_types=[
          pltpu.VMEM((2, _BPT), jnp.int32),
          pltpu.VMEM((_BPT,), jnp.int32),
          pltpu.VMEM((_BPT, 128), jnp.float32),
          pltpu.VMEM((_D, _BPT), jnp.float32),
          pltpu.SemaphoreType.DMA((_NCH,)),
          pltpu.SemaphoreType.DMA,
      ],
  )
  def _sc_gather(*refs):
    idx_refs = refs[0:nt]
    tab_refs = refs[nt:2 * nt]
    out_refs = refs[2 * nt:3 * nt]
    idx_v, row_v, lines, stage, sem_g, sem_i = refs[-6:]

    cid = lax.axis_index("c")
    sid = lax.axis_index("s")
    base = (cid * _NS + sid) * _BPT
    lanes = lax.iota(jnp.int32, 16)

    pltpu.async_copy(idx_refs[0].at[pl.ds(base, _BPT)], idx_v.at[0], sem_i)

    def do_table(t):
        p = t % 2
        pltpu.make_async_copy(
            idx_refs[t].at[pl.ds(base, _BPT)], idx_v.at[p], sem_i).wait()

        def rid(k, _):
            iv = idx_v[p, pl.ds(k * 16, 16)]
            row_v[pl.ds(k * 16, 16)] = iv >> 3
            return ()
        lax.fori_loop(0, _BPT // 16, rid, (), unroll=False)

        def gfire(j, _):
            pltpu.async_copy(
                tab_refs[t].at[row_v.at[pl.ds(j * _CH, _CH)]],
                lines.at[pl.ds(j * _CH, _CH), :],
                sem_g.at[j],
            )
            return ()
        lax.fori_loop(0, _NCH, gfire, (), unroll=False)

        if t + 1 < nt:
            pltpu.async_copy(
                idx_refs[t + 1].at[pl.ds(base, _BPT)], idx_v.at[1 - p], sem_i)

        def chunk(j, _):
            pltpu.make_async_copy(
                tab_refs[t].at[row_v.at[pl.ds(0, _CH)]],
                lines.at[pl.ds(0, _CH), :],
                sem_g.at[j],
            ).wait()

            def extract(gg, _):
                g = j * (_CH // 16) + gg
                iv = idx_v[p, pl.ds(g * 16, 16)]
                rows16 = g * 16 + lanes
                offs = (iv & 7) * 16

                def per_d(d, _):
                    vd = plsc.load_gather(lines, [rows16, offs + d])
                    stage[d, pl.ds(g * 16, 16)] = vd
                    return ()
                lax.fori_loop(0, _D, per_d, (), unroll=False)
                return ()
            lax.fori_loop(0, _CH // 16, extract, (), unroll=False)
            return ()
        lax.fori_loop(0, _NCH, chunk, (), unroll=False)

        pltpu.sync_copy(stage, out_refs[t].at[:, pl.ds(base, _BPT)])

    for t in range(nt):
        do_table(t)
  return _sc_gather


_sc_a = _make_sc_gather(13)
_sc_b = _make_sc_gather(13)


_BLK = 2048


def _tc_body(*refs):
    feats = refs[0:_NT]
    (dn0_ref, dn1_ref, wn0_ref, wn1_ref,
     w0_ref, w0n_ref, b0_ref, w1_ref, b1_ref, w2_ref, b2_ref,
     ww_ref, wnw_ref, wh_ref, ob_ref, out_ref) = refs[_NT:]
    dnums = (((0,), (0,)), ((), ()))
    h = b0_ref[:] + dn0_ref[:] * w0n_ref[0:1, :] + dn1_ref[:] * w0n_ref[1:2, :]
    for t in range(_NDC):
        h = h + lax.dot_general(feats[t][:], w0_ref[pl.ds(t * _D, _D), :],
                                dnums, preferred_element_type=jnp.float32)
    h = jax.nn.relu(h)
    h = jax.nn.relu(
        jnp.dot(h, w1_ref[:], preferred_element_type=jnp.float32) + b1_ref[:])
    h = jax.nn.relu(
        jnp.dot(h, w2_ref[:], preferred_element_type=jnp.float32) + b2_ref[:])
    acc = jnp.sum(h * wh_ref[:], axis=1, keepdims=True)
    for t in range(_NWC):
        acc = acc + lax.dot_general(feats[_NDC + t][:],
                                    ww_ref[pl.ds(t * _D, _D), :],
                                    dnums, preferred_element_type=jnp.float32)
    acc = acc + wn0_ref[:] * wnw_ref[:, 0:1] + wn1_ref[:] * wnw_ref[:, 1:2]
    out_ref[:] = jax.nn.sigmoid(acc + ob_ref[:])


def _tc_forward(feats, dn0, dn1, wn0, wn1,
                w0e, w0n, b0, w1, b1, w2, b2, ww, wnw, wh, ob):
    grid = _B // _BLK
    colT = lambda i: (0, i)
    row = lambda i: (i, 0)
    rep = lambda i: (0, 0)
    return pl.pallas_call(
        _tc_body,
        grid=(grid,),
        in_specs=(
            [pl.BlockSpec((_D, _BLK), colT) for _ in range(_NT)] +
            [
                pl.BlockSpec((_BLK, 1), row),
                pl.BlockSpec((_BLK, 1), row),
                pl.BlockSpec((_BLK, 1), row),
                pl.BlockSpec((_BLK, 1), row),
                pl.BlockSpec((_NDC * _D, 128), rep),
                pl.BlockSpec((2, 128), rep),
                pl.BlockSpec((1, 128), rep),
                pl.BlockSpec((128, 64), rep),
                pl.BlockSpec((1, 64), rep),
                pl.BlockSpec((64, 32), rep),
                pl.BlockSpec((1, 32), rep),
                pl.BlockSpec((_NWC * _D, 1), rep),
                pl.BlockSpec((1, 2), rep),
                pl.BlockSpec((1, 32), rep),
                pl.BlockSpec((1, 1), rep),
            ]
        ),
        out_specs=pl.BlockSpec((_BLK, 1), row),
        out_shape=jax.ShapeDtypeStruct((_B, 1), jnp.float32),
    )(*feats, dn0, dn1, wn0, wn1,
      w0e, w0n, b0, w1, b1, w2, b2, ww, wnw, wh, ob)


def kernel(dcat0, dcat1, dcat2, dcat3, dcat4, dcat5, dcat6, dcat7, dcat8, dcat9, dcat10, dcat11, dcat12, dcat13, dcat14, dcat15, dcat16, dcat17, dcat18, dcat19, dnum0, dnum1, wcat0, wcat1, wcat2, wcat3, wcat4, wcat5, wnum0, wnum1, deep_emb_0, deep_emb_1, deep_emb_2, deep_emb_3, deep_emb_4, deep_emb_5, deep_emb_6, deep_emb_7, deep_emb_8, deep_emb_9, deep_emb_10, deep_emb_11, deep_emb_12, deep_emb_13, deep_emb_14, deep_emb_15, deep_emb_16, deep_emb_17, deep_emb_18, deep_emb_19, wide_emb_0, wide_emb_1, wide_emb_2, wide_emb_3, wide_emb_4, wide_emb_5, mlp_W0, mlp_b0, mlp_W1, mlp_b1, mlp_W2, mlp_b2, out_W, out_b):
    idx = (dcat0, dcat1, dcat2, dcat3, dcat4, dcat5, dcat6, dcat7, dcat8,
           dcat9, dcat10, dcat11, dcat12, dcat13, dcat14, dcat15, dcat16,
           dcat17, dcat18, dcat19, wcat0, wcat1, wcat2, wcat3, wcat4, wcat5)
    tabs = (deep_emb_0, deep_emb_1, deep_emb_2, deep_emb_3, deep_emb_4,
            deep_emb_5, deep_emb_6, deep_emb_7, deep_emb_8, deep_emb_9,
            deep_emb_10, deep_emb_11, deep_emb_12, deep_emb_13, deep_emb_14,
            deep_emb_15, deep_emb_16, deep_emb_17, deep_emb_18, deep_emb_19,
            wide_emb_0, wide_emb_1, wide_emb_2, wide_emb_3, wide_emb_4,
            wide_emb_5)
    tabs_f = tuple(jax.lax.reshape(t, (_VL, 128)) for t in tabs)
    feats = _sc_a(*idx[:13], *tabs_f[:13]) + _sc_b(*idx[13:], *tabs_f[13:])

    w0e = mlp_W0[:_NDC * _D]
    w0n = mlp_W0[_NDC * _D:]
    b0 = mlp_b0.reshape(1, -1)
    b1 = mlp_b1.reshape(1, -1)
    b2 = mlp_b2.reshape(1, -1)
    ww = out_W[:_NWC * _D]
    wnw = out_W[_NWC * _D:_NWC * _D + 2].reshape(1, 2)
    wh = out_W[_NWC * _D + 2:].reshape(1, 32)
    ob = out_b.reshape(1, 1)
    return _tc_forward(feats, dnum0, dnum1, wnum0, wnum1,
                       w0e, w0n, b0, mlp_W1, b1, mlp_W2, b2, ww, wnw, wh, ob)


# speedup vs baseline: 2.8012x; 1.0407x over previous
import functools

import jax
import jax.numpy as jnp
from jax import lax
from jax.experimental import pallas as pl
from jax.experimental.pallas import tpu as pltpu
from jax.experimental.pallas import tpu_sc as plsc

_B = 16384
_V = 100000
_D = 16
_NDC = 20
_NWC = 6
_NT = _NDC + _NWC
_VL = _V * _D // 128

_NC = 2
_NS = 16
_BPT = _B // (_NC * _NS)
_CH = 128
_NCH = _BPT // _CH

_sc_mesh = plsc.VectorSubcoreMesh(
    core_axis_name="c", subcore_axis_name="s",
    num_cores=_NC, num_subcores=_NS,
)


def _make_sc_gather(nt):
  @functools.partial(
      pl.kernel,
      out_type=tuple(
          jax.ShapeDtypeStruct((_D, _B), jnp.float32) for _ in range(nt)
      ),
      mesh=_sc_mesh,
      compiler_params=pltpu.CompilerParams(needs_layout_passes=False),
      scratch_types=[
          pltpu.VMEM((2, _BPT), jnp.int32),
          pltpu.VMEM((_BPT,), jnp.int32),
          pltpu.VMEM((_BPT, 128), jnp.float32),
          pltpu.VMEM((_D, _BPT), jnp.float32),
          pltpu.SemaphoreType.DMA((_NCH,)),
          pltpu.SemaphoreType.DMA,
      ],
  )
  def _sc_gather(*refs):
    idx_refs = refs[0:nt]
    tab_refs = refs[nt:2 * nt]
    out_refs = refs[2 * nt:3 * nt]
    idx_v, row_v, lines, stage, sem_g, sem_i = refs[-6:]

    cid = lax.axis_index("c")
    sid = lax.axis_index("s")
    base = (cid * _NS + sid) * _BPT
    lanes = lax.iota(jnp.int32, 16)

    pltpu.async_copy(idx_refs[0].at[pl.ds(base, _BPT)], idx_v.at[0], sem_i)

    def do_table(t):
        p = t % 2
        pltpu.make_async_copy(
            idx_refs[t].at[pl.ds(base, _BPT)], idx_v.at[p], sem_i).wait()

        def rid(k, _):
            iv = idx_v[p, pl.ds(k * 16, 16)]
            row_v[pl.ds(k * 16, 16)] = iv >> 3
            return ()
        lax.fori_loop(0, _BPT // 16, rid, (), unroll=False)

        def gfire(j, _):
            pltpu.async_copy(
                tab_refs[t].at[row_v.at[pl.ds(j * _CH, _CH)]],
                lines.at[pl.ds(j * _CH, _CH), :],
                sem_g.at[j],
            )
            return ()
        lax.fori_loop(0, _NCH, gfire, (), unroll=False)

        if t + 1 < nt:
            pltpu.async_copy(
                idx_refs[t + 1].at[pl.ds(base, _BPT)], idx_v.at[1 - p], sem_i)

        def chunk(j, _):
            pltpu.make_async_copy(
                tab_refs[t].at[row_v.at[pl.ds(0, _CH)]],
                lines.at[pl.ds(0, _CH), :],
                sem_g.at[j],
            ).wait()

            def extract(gg, _):
                g = j * (_CH // 16) + gg
                iv = idx_v[p, pl.ds(g * 16, 16)]
                rows16 = g * 16 + lanes
                offs = (iv & 7) * 16

                def per_d(d, _):
                    vd = plsc.load_gather(lines, [rows16, offs + d])
                    stage[d, pl.ds(g * 16, 16)] = vd
                    return ()
                lax.fori_loop(0, _D, per_d, (), unroll=False)
                return ()
            lax.fori_loop(0, _CH // 16, extract, (), unroll=False)
            return ()
        lax.fori_loop(0, _NCH, chunk, (), unroll=False)

        pltpu.sync_copy(stage, out_refs[t].at[:, pl.ds(base, _BPT)])

    for t in range(nt):
        do_table(t)
  return _sc_gather


_sc_parts = (7, 7, 6, 6)
_sc_fns = tuple(_make_sc_gather(n) for n in _sc_parts)


_BLK = 2048


def _tc_body(*refs):
    feats = refs[0:_NT]
    (dn0_ref, dn1_ref, wn0_ref, wn1_ref,
     w0_ref, w0n_ref, b0_ref, w1_ref, b1_ref, w2_ref, b2_ref,
     ww_ref, wnw_ref, wh_ref, ob_ref, out_ref) = refs[_NT:]
    dnums = (((0,), (0,)), ((), ()))
    h = b0_ref[:] + dn0_ref[:] * w0n_ref[0:1, :] + dn1_ref[:] * w0n_ref[1:2, :]
    for t in range(_NDC):
        h = h + lax.dot_general(feats[t][:], w0_ref[pl.ds(t * _D, _D), :],
                                dnums, preferred_element_type=jnp.float32)
    h = jax.nn.relu(h)
    h = jax.nn.relu(
        jnp.dot(h, w1_ref[:], preferred_element_type=jnp.float32) + b1_ref[:])
    h = jax.nn.relu(
        jnp.dot(h, w2_ref[:], preferred_element_type=jnp.float32) + b2_ref[:])
    acc = jnp.sum(h * wh_ref[:], axis=1, keepdims=True)
    for t in range(_NWC):
        acc = acc + lax.dot_general(feats[_NDC + t][:],
                                    ww_ref[pl.ds(t * _D, _D), :],
                                    dnums, preferred_element_type=jnp.float32)
    acc = acc + wn0_ref[:] * wnw_ref[:, 0:1] + wn1_ref[:] * wnw_ref[:, 1:2]
    out_ref[:] = jax.nn.sigmoid(acc + ob_ref[:])


def _tc_forward(feats, dn0, dn1, wn0, wn1,
                w0e, w0n, b0, w1, b1, w2, b2, ww, wnw, wh, ob):
    grid = _B // _BLK
    colT = lambda i: (0, i)
    row = lambda i: (i, 0)
    rep = lambda i: (0, 0)
    return pl.pallas_call(
        _tc_body,
        grid=(grid,),
        in_specs=(
            [pl.BlockSpec((_D, _BLK), colT) for _ in range(_NT)] +
            [
                pl.BlockSpec((_BLK, 1), row),
                pl.BlockSpec((_BLK, 1), row),
                pl.BlockSpec((_BLK, 1), row),
                pl.BlockSpec((_BLK, 1), row),
                pl.BlockSpec((_NDC * _D, 128), rep),
                pl.BlockSpec((2, 128), rep),
                pl.BlockSpec((1, 128), rep),
                pl.BlockSpec((128, 64), rep),
                pl.BlockSpec((1, 64), rep),
                pl.BlockSpec((64, 32), rep),
                pl.BlockSpec((1, 32), rep),
                pl.BlockSpec((_NWC * _D, 1), rep),
                pl.BlockSpec((1, 2), rep),
                pl.BlockSpec((1, 32), rep),
                pl.BlockSpec((1, 1), rep),
            ]
        ),
        out_specs=pl.BlockSpec((_BLK, 1), row),
        out_shape=jax.ShapeDtypeStruct((_B, 1), jnp.float32),
    )(*feats, dn0, dn1, wn0, wn1,
      w0e, w0n, b0, w1, b1, w2, b2, ww, wnw, wh, ob)


def kernel(dcat0, dcat1, dcat2, dcat3, dcat4, dcat5, dcat6, dcat7, dcat8, dcat9, dcat10, dcat11, dcat12, dcat13, dcat14, dcat15, dcat16, dcat17, dcat18, dcat19, dnum0, dnum1, wcat0, wcat1, wcat2, wcat3, wcat4, wcat5, wnum0, wnum1, deep_emb_0, deep_emb_1, deep_emb_2, deep_emb_3, deep_emb_4, deep_emb_5, deep_emb_6, deep_emb_7, deep_emb_8, deep_emb_9, deep_emb_10, deep_emb_11, deep_emb_12, deep_emb_13, deep_emb_14, deep_emb_15, deep_emb_16, deep_emb_17, deep_emb_18, deep_emb_19, wide_emb_0, wide_emb_1, wide_emb_2, wide_emb_3, wide_emb_4, wide_emb_5, mlp_W0, mlp_b0, mlp_W1, mlp_b1, mlp_W2, mlp_b2, out_W, out_b):
    idx = (dcat0, dcat1, dcat2, dcat3, dcat4, dcat5, dcat6, dcat7, dcat8,
           dcat9, dcat10, dcat11, dcat12, dcat13, dcat14, dcat15, dcat16,
           dcat17, dcat18, dcat19, wcat0, wcat1, wcat2, wcat3, wcat4, wcat5)
    tabs = (deep_emb_0, deep_emb_1, deep_emb_2, deep_emb_3, deep_emb_4,
            deep_emb_5, deep_emb_6, deep_emb_7, deep_emb_8, deep_emb_9,
            deep_emb_10, deep_emb_11, deep_emb_12, deep_emb_13, deep_emb_14,
            deep_emb_15, deep_emb_16, deep_emb_17, deep_emb_18, deep_emb_19,
            wide_emb_0, wide_emb_1, wide_emb_2, wide_emb_3, wide_emb_4,
            wide_emb_5)
    tabs_f = tuple(jax.lax.reshape(t, (_VL, 128)) for t in tabs)
    feats = ()
    off = 0
    for n, fn in zip(_sc_parts, _sc_fns):
        feats = feats + fn(*idx[off:off + n], *tabs_f[off:off + n])
        off += n

    w0e = mlp_W0[:_NDC * _D]
    w0n = mlp_W0[_NDC * _D:]
    b0 = mlp_b0.reshape(1, -1)
    b1 = mlp_b1.reshape(1, -1)
    b2 = mlp_b2.reshape(1, -1)
    ww = out_W[:_NWC * _D]
    wnw = out_W[_NWC * _D:_NWC * _D + 2].reshape(1, 2)
    wh = out_W[_NWC * _D + 2:].reshape(1, 32)
    ob = out_b.reshape(1, 1)
    return _tc_forward(feats, dnum0, dnum1, wnum0, wnum1,
                       w0e, w0n, b0, mlp_W1, b1, mlp_W2, b2, ww, wnw, wh, ob)


# speedup vs baseline: 2.8448x; 1.0156x over previous
import functools

import jax
import jax.numpy as jnp
from jax import lax
from jax.experimental import pallas as pl
from jax.experimental.pallas import tpu as pltpu
from jax.experimental.pallas import tpu_sc as plsc

_B = 16384
_V = 100000
_D = 16
_NDC = 20
_NWC = 6
_NT = _NDC + _NWC
_VL = _V * _D // 128

_NC = 2
_NS = 16
_BPT = _B // (_NC * _NS)
_CH = 128
_NCH = _BPT // _CH

_sc_mesh = plsc.VectorSubcoreMesh(
    core_axis_name="c", subcore_axis_name="s",
    num_cores=_NC, num_subcores=_NS,
)


def _make_sc_gather(nt):
  @functools.partial(
      pl.kernel,
      out_type=tuple(
          jax.ShapeDtypeStruct((_D, _B), jnp.float32) for _ in range(nt)
      ),
      mesh=_sc_mesh,
      compiler_params=pltpu.CompilerParams(needs_layout_passes=False),
      scratch_types=[
          pltpu.VMEM((2, _BPT), jnp.int32),
          pltpu.VMEM((_BPT,), jnp.int32),
          pltpu.VMEM((_BPT, 128), jnp.float32),
          pltpu.VMEM((_D, _BPT), jnp.float32),
          pltpu.SemaphoreType.DMA((_NCH,)),
          pltpu.SemaphoreType.DMA,
      ],
  )
  def _sc_gather(*refs):
    idx_refs = refs[0:nt]
    tab_refs = refs[nt:2 * nt]
    out_refs = refs[2 * nt:3 * nt]
    idx_v, row_v, lines, stage, sem_g, sem_i = refs[-6:]

    cid = lax.axis_index("c")
    sid = lax.axis_index("s")
    base = (cid * _NS + sid) * _BPT
    lanes = lax.iota(jnp.int32, 16)

    pltpu.async_copy(idx_refs[0].at[pl.ds(base, _BPT)], idx_v.at[0], sem_i)

    def do_table(t):
        p = t % 2
        pltpu.make_async_copy(
            idx_refs[t].at[pl.ds(base, _BPT)], idx_v.at[p], sem_i).wait()

        def rid(k, _):
            iv = idx_v[p, pl.ds(k * 16, 16)]
            row_v[pl.ds(k * 16, 16)] = iv >> 3
            return ()
        lax.fori_loop(0, _BPT // 16, rid, (), unroll=False)

        def gfire(j, _):
            pltpu.async_copy(
                tab_refs[t].at[row_v.at[pl.ds(j * _CH, _CH)]],
                lines.at[pl.ds(j * _CH, _CH), :],
                sem_g.at[j],
            )
            return ()
        lax.fori_loop(0, _NCH, gfire, (), unroll=False)

        if t + 1 < nt:
            pltpu.async_copy(
                idx_refs[t + 1].at[pl.ds(base, _BPT)], idx_v.at[1 - p], sem_i)

        def chunk(j, _):
            pltpu.make_async_copy(
                tab_refs[t].at[row_v.at[pl.ds(0, _CH)]],
                lines.at[pl.ds(0, _CH), :],
                sem_g.at[j],
            ).wait()

            def extract(gg, _):
                g = j * (_CH // 16) + gg
                iv = idx_v[p, pl.ds(g * 16, 16)]
                rows16 = g * 16 + lanes
                offs = (iv & 7) * 16

                def per_d(d, _):
                    vd = plsc.load_gather(lines, [rows16, offs + d])
                    stage[d, pl.ds(g * 16, 16)] = vd
                    return ()
                lax.fori_loop(0, _D, per_d, (), unroll=False)
                return ()
            lax.fori_loop(0, _CH // 16, extract, (), unroll=False)
            return ()
        lax.fori_loop(0, _NCH, chunk, (), unroll=False)

        pltpu.sync_copy(stage, out_refs[t].at[:, pl.ds(base, _BPT)])

    for t in range(nt):
        do_table(t)
  return _sc_gather


_sc_parts = (4, 4, 4, 4, 4, 3, 3)
_sc_fns = tuple(_make_sc_gather(n) for n in _sc_parts)


_BLK = 2048


def _tc_body(*refs):
    feats = refs[0:_NT]
    (dn0_ref, dn1_ref, wn0_ref, wn1_ref,
     w0_ref, w0n_ref, b0_ref, w1_ref, b1_ref, w2_ref, b2_ref,
     ww_ref, wnw_ref, wh_ref, ob_ref, out_ref) = refs[_NT:]
    dnums = (((0,), (0,)), ((), ()))
    h = b0_ref[:] + dn0_ref[:] * w0n_ref[0:1, :] + dn1_ref[:] * w0n_ref[1:2, :]
    for t in range(_NDC):
        h = h + lax.dot_general(feats[t][:], w0_ref[pl.ds(t * _D, _D), :],
                                dnums, preferred_element_type=jnp.float32)
    h = jax.nn.relu(h)
    h = jax.nn.relu(
        jnp.dot(h, w1_ref[:], preferred_element_type=jnp.float32) + b1_ref[:])
    h = jax.nn.relu(
        jnp.dot(h, w2_ref[:], preferred_element_type=jnp.float32) + b2_ref[:])
    acc = jnp.sum(h * wh_ref[:], axis=1, keepdims=True)
    for t in range(_NWC):
        acc = acc + lax.dot_general(feats[_NDC + t][:],
                                    ww_ref[pl.ds(t * _D, _D), :],
                                    dnums, preferred_element_type=jnp.float32)
    acc = acc + wn0_ref[:] * wnw_ref[:, 0:1] + wn1_ref[:] * wnw_ref[:, 1:2]
    out_ref[:] = jax.nn.sigmoid(acc + ob_ref[:])


def _tc_forward(feats, dn0, dn1, wn0, wn1,
                w0e, w0n, b0, w1, b1, w2, b2, ww, wnw, wh, ob):
    grid = _B // _BLK
    colT = lambda i: (0, i)
    row = lambda i: (i, 0)
    rep = lambda i: (0, 0)
    return pl.pallas_call(
        _tc_body,
        grid=(grid,),
        in_specs=(
            [pl.BlockSpec((_D, _BLK), colT) for _ in range(_NT)] +
            [
                pl.BlockSpec((_BLK, 1), row),
                pl.BlockSpec((_BLK, 1), row),
                pl.BlockSpec((_BLK, 1), row),
                pl.BlockSpec((_BLK, 1), row),
                pl.BlockSpec((_NDC * _D, 128), rep),
                pl.BlockSpec((2, 128), rep),
                pl.BlockSpec((1, 128), rep),
                pl.BlockSpec((128, 64), rep),
                pl.BlockSpec((1, 64), rep),
                pl.BlockSpec((64, 32), rep),
                pl.BlockSpec((1, 32), rep),
                pl.BlockSpec((_NWC * _D, 1), rep),
                pl.BlockSpec((1, 2), rep),
                pl.BlockSpec((1, 32), rep),
                pl.BlockSpec((1, 1), rep),
            ]
        ),
        out_specs=pl.BlockSpec((_BLK, 1), row),
        out_shape=jax.ShapeDtypeStruct((_B, 1), jnp.float32),
    )(*feats, dn0, dn1, wn0, wn1,
      w0e, w0n, b0, w1, b1, w2, b2, ww, wnw, wh, ob)


def kernel(dcat0, dcat1, dcat2, dcat3, dcat4, dcat5, dcat6, dcat7, dcat8, dcat9, dcat10, dcat11, dcat12, dcat13, dcat14, dcat15, dcat16, dcat17, dcat18, dcat19, dnum0, dnum1, wcat0, wcat1, wcat2, wcat3, wcat4, wcat5, wnum0, wnum1, deep_emb_0, deep_emb_1, deep_emb_2, deep_emb_3, deep_emb_4, deep_emb_5, deep_emb_6, deep_emb_7, deep_emb_8, deep_emb_9, deep_emb_10, deep_emb_11, deep_emb_12, deep_emb_13, deep_emb_14, deep_emb_15, deep_emb_16, deep_emb_17, deep_emb_18, deep_emb_19, wide_emb_0, wide_emb_1, wide_emb_2, wide_emb_3, wide_emb_4, wide_emb_5, mlp_W0, mlp_b0, mlp_W1, mlp_b1, mlp_W2, mlp_b2, out_W, out_b):
    idx = (dcat0, dcat1, dcat2, dcat3, dcat4, dcat5, dcat6, dcat7, dcat8,
           dcat9, dcat10, dcat11, dcat12, dcat13, dcat14, dcat15, dcat16,
           dcat17, dcat18, dcat19, wcat0, wcat1, wcat2, wcat3, wcat4, wcat5)
    tabs = (deep_emb_0, deep_emb_1, deep_emb_2, deep_emb_3, deep_emb_4,
            deep_emb_5, deep_emb_6, deep_emb_7, deep_emb_8, deep_emb_9,
            deep_emb_10, deep_emb_11, deep_emb_12, deep_emb_13, deep_emb_14,
            deep_emb_15, deep_emb_16, deep_emb_17, deep_emb_18, deep_emb_19,
            wide_emb_0, wide_emb_1, wide_emb_2, wide_emb_3, wide_emb_4,
            wide_emb_5)
    tabs_f = tuple(jax.lax.reshape(t, (_VL, 128)) for t in tabs)
    feats = ()
    off = 0
    for n, fn in zip(_sc_parts, _sc_fns):
        feats = feats + fn(*idx[off:off + n], *tabs_f[off:off + n])
        off += n

    w0e = mlp_W0[:_NDC * _D]
    w0n = mlp_W0[_NDC * _D:]
    b0 = mlp_b0.reshape(1, -1)
    b1 = mlp_b1.reshape(1, -1)
    b2 = mlp_b2.reshape(1, -1)
    ww = out_W[:_NWC * _D]
    wnw = out_W[_NWC * _D:_NWC * _D + 2].reshape(1, 2)
    wh = out_W[_NWC * _D + 2:].reshape(1, 32)
    ob = out_b.reshape(1, 1)
    return _tc_forward(feats, dnum0, dnum1, wnum0, wnum1,
                       w0e, w0n, b0, mlp_W1, b1, mlp_W2, b2, ww, wnw, wh, ob)
